# baseline (speedup 1.0000x reference)
import functools

import jax
import jax.numpy as jnp
from jax import lax
from jax.experimental import pallas as pl
from jax.experimental.pallas import tpu as pltpu
from jax.experimental.pallas import tpu_sc as plsc

FM_H = 496
FM_W = 432
HW = FM_H * FM_W
B = 4
C = 64
P = 12000

NSEG = 8
SEG_COLS = FM_W // NSEG
SEG_CELLS = SEG_COLS * FM_H
NCHUNK = P // 16

CQ = 4
NUNIT_PER_TILE = (B * C // CQ) // 32
STRIPE_COLS = 8
NSTRIPE = FM_W // STRIPE_COLS
STRIPE_CELLS = STRIPE_COLS * FM_H
STRIPE_VECS = STRIPE_CELLS // 16
STRIPE_COLS_LOG2 = 3


def _wid():
    return lax.axis_index("s") * 2 + lax.axis_index("c")


def _build_map_body(flag_hbm, xi_hbm, yi_hbm, pmap_hbm, flagv, xiv, yiv, mapv):
    wid = _wid()
    b = wid // NSEG
    seg = wid % NSEG
    x0 = seg * SEG_COLS

    poff = pl.multiple_of(b * P, 8)
    pltpu.sync_copy(flag_hbm.at[pl.ds(poff, P)], flagv)
    pltpu.sync_copy(xi_hbm.at[pl.ds(poff, P)], xiv)
    pltpu.sync_copy(yi_hbm.at[pl.ds(poff, P)], yiv)

    neg1 = jnp.full((16,), -1, jnp.int32)

    def init_body(i, _):
        mapv[pl.ds(i * 16, 16)] = neg1
        return 0

    lax.fori_loop(0, SEG_CELLS // 16, init_body, 0)

    lanes = lax.iota(jnp.int32, 16)
    lane_masks = [lanes == jnp.full((16,), l, jnp.int32) for l in range(16)]

    def chunk_body(k, _):
        fl = flagv[pl.ds(k * 16, 16)]
        xv = xiv[pl.ds(k * 16, 16)]
        yv = yiv[pl.ds(k * 16, 16)]
        xl = xv - jnp.full((16,), 1, jnp.int32) * x0
        m = (fl == jnp.full((16,), 1, jnp.int32)) \
            & (xl >= jnp.full((16,), 0, jnp.int32)) \
            & (xl < jnp.full((16,), SEG_COLS, jnp.int32))
        cell = xl * jnp.full((16,), FM_H, jnp.int32) + yv
        cell = jnp.where(m, cell, jnp.full((16,), 0, jnp.int32))
        pvec = lanes + jnp.full((16,), 16, jnp.int32) * k
        for lm in lane_masks:
            plsc.store_scatter(mapv, [cell], pvec, mask=m & lm)
        return 0

    lax.fori_loop(0, NCHUNK, chunk_body, 0)

    moff = pl.multiple_of(b * HW + x0 * FM_H, 8)
    pltpu.sync_copy(mapv, pmap_hbm.at[pl.ds(moff, SEG_CELLS)])


def _compose_body(
    x_hbm, pmap_hbm, out_hbm, xv, mapv, outv, lref, sm0, sm1, so0, so1
):
    wid = _wid()
    zero16 = jnp.full((16,), 0, jnp.int32)
    zf16 = jnp.full((16,), 0.0, jnp.float32)
    sm = (sm0, sm1)
    so = (so0, so1)

    def map_slice(b, s):
        soff = pl.multiple_of(b * HW + s * STRIPE_CELLS, 8)
        return pmap_hbm.at[pl.ds(soff, STRIPE_CELLS)]

    def out_slice(b, c0, c, s):
        row0 = pl.multiple_of((b * C + c0 + c) * FM_W + s * STRIPE_COLS, 8)
        return out_hbm.at[pl.ds(row0, STRIPE_COLS), :]

    lanes = lax.iota(jnp.int32, 16)
    lanes_sh = lax.shift_left(lanes, jnp.full((16,), 14, jnp.int32))

    def apply_list(par, cntv, nv, c_list, vals_fn):
        par16 = jnp.full((16,), par, jnp.int32)

        def jbody(j, _):
            pk = lref[pl.ds(par * STRIPE_CELLS + j * 16, 16)]
            msk = (jnp.full((16,), 16, jnp.int32) * j + lanes) < cntv
            wv = lax.shift_right_logical(pk, jnp.full((16,), 26, jnp.int32))
            hv = lax.shift_right_logical(pk, jnp.full((16,), 14, jnp.int32)) \
                & jnp.full((16,), 511, jnp.int32)
            pv = pk & jnp.full((16,), 16383, jnp.int32)
            for c in c_list:
                plsc.store_scatter(
                    outv,
                    [par16, jnp.full((16,), c, jnp.int32), wv, hv],
                    vals_fn(c, pv, msk),
                    mask=msk,
                )
            return 0

        lax.fori_loop(0, nv, jbody, 0)

    for u in range(NUNIT_PER_TILE):
        g = wid * NUNIT_PER_TILE + u
        b = g // (C // CQ)
        c0 = (g % (C // CQ)) * CQ

        xoff = pl.multiple_of((b * C + c0) * P, 8)
        pltpu.sync_copy(x_hbm.at[pl.ds(xoff, CQ * P)], xv)

        def sp_body(sp, carry):
            cnts = list(carry)
            for par in range(2):
                s = 2 * sp + par
                pltpu.make_async_copy(
                    map_slice(b, s), mapv.at[par], sm[par]
                ).wait()

                @pl.when(sp > 0)
                def _():
                    for c in range(CQ):
                        pltpu.make_async_copy(
                            outv.at[par, c],
                            out_slice(b, c0, c, s),
                            so[par],
                        ).wait()

                nv_old = lax.shift_right_logical(
                    jnp.max(cnts[par]) + 15, 4
                )

                @pl.when(sp > 0)
                def _():
                    apply_list(
                        par, cnts[par], nv_old, range(CQ),
                        lambda c, pv, msk: zf16,
                    )

                @plsc.parallel_loop(
                    0, STRIPE_VECS, 1, unroll=4, carry=jnp.zeros((16,), jnp.int32)
                )
                def cbody(v, base):
                    w = v // (FM_H // 16)
                    hb = (v - w * (FM_H // 16)) * 16
                    m16 = mapv[par, pl.ds(v * 16, 16)]
                    msk = m16 >= zero16
                    mi = jnp.where(msk, jnp.full((16,), 1, jnp.int32), zero16)
                    pos = plsc.cumsum(mi) - jnp.full((16,), 1, jnp.int32) + base
                    packed = (
                        jnp.full((16,), 1, jnp.int32) * ((w * 67108864) + (hb * 16384))
                        + lanes_sh + m16
                    )
                    plsc.store_scatter(
                        lref,
                        [jnp.full((16,), par * STRIPE_CELLS, jnp.int32) + pos],
                        packed,
                        mask=msk,
                    )
                    return base + plsc.all_reduce_population_count(msk)

                cnts[par] = cbody
                nv_new = lax.shift_right_logical(jnp.max(cnts[par]) + 15, 4)

                apply_list(
                    par, cnts[par], nv_new, range(CQ),
                    lambda c, pv, msk: plsc.load_gather(
                        xv,
                        [jnp.full((16,), c * P, jnp.int32) + pv],
                        mask=msk,
                    ),
                )

                for c in range(CQ):
                    pltpu.async_copy(
                        outv.at[par, c],
                        out_slice(b, c0, c, s),
                        so[par],
                    )

                @pl.when(s + 2 < NSTRIPE)
                def _():
                    pltpu.async_copy(
                        map_slice(b, s + 2), mapv.at[par], sm[par]
                    )

            return tuple(cnts)

        for par in range(2):
            for c in range(CQ):
                @plsc.parallel_loop(0, STRIPE_VECS, 1, unroll=4)
                def _(v):
                    w = v // (FM_H // 16)
                    hb = (v - w * (FM_H // 16)) * 16
                    outv[par, c, w, pl.ds(hb, 16)] = zf16

        for par in range(2):
            pltpu.async_copy(map_slice(b, par), mapv.at[par], sm[par])

        zc = jnp.zeros((16,), jnp.int32)
        lax.fori_loop(0, NSTRIPE // 2, sp_body, (zc, zc))

        for par in range(2):
            s_last = NSTRIPE - 2 + par
            for c in range(CQ):
                pltpu.make_async_copy(
                    outv.at[par, c],
                    out_slice(b, c0, c, s_last),
                    so[par],
                ).wait()


@functools.lru_cache(maxsize=1)
def _kernels():
    mesh = plsc.VectorSubcoreMesh(
        core_axis_name="c", subcore_axis_name="s", num_cores=2, num_subcores=16
    )
    params = pltpu.CompilerParams(needs_layout_passes=False)
    build_map = pl.kernel(
        _build_map_body,
        out_type=jax.ShapeDtypeStruct((B * HW,), jnp.int32),
        mesh=mesh,
        compiler_params=params,
        scratch_types=[
            pltpu.VMEM((P,), jnp.int32),
            pltpu.VMEM((P,), jnp.int32),
            pltpu.VMEM((P,), jnp.int32),
            pltpu.VMEM((SEG_CELLS,), jnp.int32),
        ],
    )
    compose = pl.kernel(
        _compose_body,
        out_type=jax.ShapeDtypeStruct((B * C * FM_W, FM_H), jnp.float32),
        mesh=mesh,
        compiler_params=params,
        scratch_types=[
            pltpu.VMEM((CQ * P,), jnp.float32),
            pltpu.VMEM((2, STRIPE_CELLS), jnp.int32),
            pltpu.VMEM((2, CQ, STRIPE_COLS, FM_H), jnp.float32),
            pltpu.VMEM((2 * STRIPE_CELLS,), jnp.int32),
            pltpu.SemaphoreType.DMA,
            pltpu.SemaphoreType.DMA,
            pltpu.SemaphoreType.DMA,
            pltpu.SemaphoreType.DMA,
        ],
    )
    return build_map, compose


def kernel(x, inds):
    build_map, compose = _kernels()
    flag = inds[..., 0].astype(jnp.int32).reshape(-1)
    xi = inds[..., 1].astype(jnp.int32).reshape(-1)
    yi = inds[..., 2].astype(jnp.int32).reshape(-1)
    pmap = build_map(flag, xi, yi)
    out = compose(x.reshape(-1), pmap)
    return out.reshape(B, C, FM_W, FM_H).transpose(0, 1, 3, 2)


# speedup vs baseline: 1.2180x; 1.2180x over previous
import functools

import jax
import jax.numpy as jnp
from jax import lax
from jax.experimental import pallas as pl
from jax.experimental.pallas import tpu as pltpu
from jax.experimental.pallas import tpu_sc as plsc

FM_H = 496
FM_W = 432
HW = FM_H * FM_W
B = 4
C = 64
P = 12000

NSEG = 8
SEG_COLS = FM_W // NSEG
SEG_CELLS = SEG_COLS * FM_H
NCHUNK = P // 16

CQ = 4
NUNIT_PER_TILE = (B * C // CQ) // 32
STRIPE_COLS = 8
NSTRIPE = FM_W // STRIPE_COLS
STRIPE_CELLS = STRIPE_COLS * FM_H
STRIPE_VECS = STRIPE_CELLS // 16
STRIPE_COLS_LOG2 = 3


def _wid():
    return lax.axis_index("s") * 2 + lax.axis_index("c")


def _build_map_body(
    flag_hbm, xi_hbm, yi_hbm, pmap_hbm, flagv, xiv, yiv, mapv, sst
):
    wid = _wid()
    b = wid // NSEG
    seg = wid % NSEG
    x0 = seg * SEG_COLS

    poff = pl.multiple_of(b * P, 8)
    cps = [
        pltpu.async_copy(flag_hbm.at[pl.ds(poff, P)], flagv, sst),
        pltpu.async_copy(xi_hbm.at[pl.ds(poff, P)], xiv, sst),
        pltpu.async_copy(yi_hbm.at[pl.ds(poff, P)], yiv, sst),
    ]

    neg1 = jnp.full((16,), -1, jnp.int32)

    @plsc.parallel_loop(0, SEG_CELLS // 16, 1, unroll=2)
    def _(i):
        mapv[pl.ds(i * 16, 16)] = neg1

    for cp in cps:
        cp.wait()

    lanes = lax.iota(jnp.int32, 16)
    lane_masks = [lanes == jnp.full((16,), l, jnp.int32) for l in range(16)]

    def chunk_body(k, _):
        fl = flagv[pl.ds(k * 16, 16)]
        xv = xiv[pl.ds(k * 16, 16)]
        yv = yiv[pl.ds(k * 16, 16)]
        xl = xv - jnp.full((16,), 1, jnp.int32) * x0
        m = (fl == jnp.full((16,), 1, jnp.int32)) \
            & (xl >= jnp.full((16,), 0, jnp.int32)) \
            & (xl < jnp.full((16,), SEG_COLS, jnp.int32))
        cell = xl * jnp.full((16,), FM_H, jnp.int32) + yv
        cell = jnp.where(m, cell, jnp.full((16,), 0, jnp.int32))
        pvec = lanes + jnp.full((16,), 16, jnp.int32) * k
        for lm in lane_masks:
            plsc.store_scatter(mapv, [cell], pvec, mask=m & lm)
        return 0

    lax.fori_loop(0, NCHUNK, chunk_body, 0)

    moff = pl.multiple_of(b * HW + x0 * FM_H, 8)
    pltpu.sync_copy(mapv, pmap_hbm.at[pl.ds(moff, SEG_CELLS)])


def _compose_body(
    x_hbm, pmap_hbm, out_hbm, xv, mapv, outv, sm0, sm1, so0, so1
):
    wid = _wid()
    zero16 = jnp.full((16,), 0, jnp.int32)
    zf16 = jnp.full((16,), 0.0, jnp.float32)
    sm = (sm0, sm1)
    so = (so0, so1)

    def map_slice(b, s):
        soff = pl.multiple_of(b * HW + s * STRIPE_CELLS, 8)
        return pmap_hbm.at[pl.ds(soff, STRIPE_CELLS)]

    def out_slice(b, c0, c, s):
        row0 = pl.multiple_of((b * C + c0 + c) * FM_W + s * STRIPE_COLS, 8)
        return out_hbm.at[pl.ds(row0, STRIPE_COLS), :]

    for u in range(NUNIT_PER_TILE):
        g = wid * NUNIT_PER_TILE + u
        b = g // (C // CQ)
        c0 = (g % (C // CQ)) * CQ

        for par in range(2):
            pltpu.async_copy(map_slice(b, par), mapv.at[par], sm[par])

        xoff = pl.multiple_of((b * C + c0) * P, 8)
        pltpu.sync_copy(x_hbm.at[pl.ds(xoff, CQ * P)], xv)

        def sp_body(sp, _):
            for par in range(2):
                s = 2 * sp + par
                pltpu.make_async_copy(
                    map_slice(b, s), mapv.at[par], sm[par]
                ).wait()

                @pl.when(sp > 0)
                def _():
                    for c in range(CQ):
                        pltpu.make_async_copy(
                            outv.at[par, c],
                            out_slice(b, c0, c, s),
                            so[par],
                        ).wait()

                @plsc.parallel_loop(0, STRIPE_VECS, 1, unroll=8)
                def _(v):
                    w = lax.bitwise_and(v, STRIPE_COLS - 1)
                    h0 = lax.shift_left(
                        lax.shift_right_logical(v, STRIPE_COLS_LOG2), 4
                    )
                    m16 = mapv[par, pl.ds(w * FM_H + h0, 16)]
                    msk = m16 >= zero16
                    idx = jnp.maximum(m16, zero16)
                    for c in range(CQ):
                        gvals = plsc.load_gather(
                            xv, [jnp.full((16,), c * P, jnp.int32) + idx]
                        )
                        outv[par, c, w, pl.ds(h0, 16)] = jnp.where(
                            msk, gvals, zf16
                        )

                for c in range(CQ):
                    pltpu.async_copy(
                        outv.at[par, c],
                        out_slice(b, c0, c, s),
                        so[par],
                    )

                @pl.when(s + 2 < NSTRIPE)
                def _():
                    pltpu.async_copy(
                        map_slice(b, s + 2), mapv.at[par], sm[par]
                    )

            return 0

        lax.fori_loop(0, NSTRIPE // 2, sp_body, 0)

        for par in range(2):
            s_last = NSTRIPE - 2 + par
            for c in range(CQ):
                pltpu.make_async_copy(
                    outv.at[par, c],
                    out_slice(b, c0, c, s_last),
                    so[par],
                ).wait()


@functools.lru_cache(maxsize=1)
def _kernels():
    mesh = plsc.VectorSubcoreMesh(
        core_axis_name="c", subcore_axis_name="s", num_cores=2, num_subcores=16
    )
    params = pltpu.CompilerParams(needs_layout_passes=False)
    build_map = pl.kernel(
        _build_map_body,
        out_type=jax.ShapeDtypeStruct((B * HW,), jnp.int32),
        mesh=mesh,
        compiler_params=params,
        scratch_types=[
            pltpu.VMEM((P,), jnp.int32),
            pltpu.VMEM((P,), jnp.int32),
            pltpu.VMEM((P,), jnp.int32),
            pltpu.VMEM((SEG_CELLS,), jnp.int32),
            pltpu.SemaphoreType.DMA,
        ],
    )
    compose = pl.kernel(
        _compose_body,
        out_type=jax.ShapeDtypeStruct((B * C * FM_W, FM_H), jnp.float32),
        mesh=mesh,
        compiler_params=params,
        scratch_types=[
            pltpu.VMEM((CQ * P,), jnp.float32),
            pltpu.VMEM((2, STRIPE_CELLS), jnp.int32),
            pltpu.VMEM((2, CQ, STRIPE_COLS, FM_H), jnp.float32),
            pltpu.SemaphoreType.DMA,
            pltpu.SemaphoreType.DMA,
            pltpu.SemaphoreType.DMA,
            pltpu.SemaphoreType.DMA,
        ],
    )
    return build_map, compose


def kernel(x, inds):
    build_map, compose = _kernels()
    flag = inds[..., 0].astype(jnp.int32).reshape(-1)
    xi = inds[..., 1].astype(jnp.int32).reshape(-1)
    yi = inds[..., 2].astype(jnp.int32).reshape(-1)
    pmap = build_map(flag, xi, yi)
    out = compose(x.reshape(-1), pmap)
    return out.reshape(B, C, FM_W, FM_H).transpose(0, 1, 3, 2)


# speedup vs baseline: 1.2416x; 1.0193x over previous
import functools

import jax
import jax.numpy as jnp
from jax import lax
from jax.experimental import pallas as pl
from jax.experimental.pallas import tpu as pltpu
from jax.experimental.pallas import tpu_sc as plsc

FM_H = 496
FM_W = 432
HW = FM_H * FM_W
B = 4
C = 64
P = 12000

NSEG = 8
SEG_COLS = FM_W // NSEG
SEG_CELLS = SEG_COLS * FM_H
NCHUNK = P // 16

CQ = 4
NUNIT_PER_TILE = (B * C // CQ) // 32
STRIPE_COLS = 8
NSTRIPE = FM_W // STRIPE_COLS
STRIPE_CELLS = STRIPE_COLS * FM_H
STRIPE_VECS = STRIPE_CELLS // 16
STRIPE_COLS_LOG2 = 3


def _wid():
    return lax.axis_index("s") * 2 + lax.axis_index("c")


def _build_map_body(
    flag_hbm, xi_hbm, yi_hbm, pmap_hbm, flagv, xiv, yiv, mapv, sst
):
    wid = _wid()
    b = wid // NSEG
    seg = wid % NSEG
    x0 = seg * SEG_COLS

    poff = pl.multiple_of(b * P, 8)
    cps = [
        pltpu.async_copy(flag_hbm.at[pl.ds(poff, P)], flagv, sst),
        pltpu.async_copy(xi_hbm.at[pl.ds(poff, P)], xiv, sst),
        pltpu.async_copy(yi_hbm.at[pl.ds(poff, P)], yiv, sst),
    ]

    neg1 = jnp.full((16,), -1, jnp.int32)

    @plsc.parallel_loop(0, SEG_CELLS // 16, 1, unroll=2)
    def _(i):
        mapv[pl.ds(i * 16, 16)] = neg1

    for cp in cps:
        cp.wait()

    lanes = lax.iota(jnp.int32, 16)
    lane_masks = [lanes == jnp.full((16,), l, jnp.int32) for l in range(16)]

    def chunk_body(k, _):
        fl = flagv[pl.ds(k * 16, 16)]
        xv = xiv[pl.ds(k * 16, 16)]
        yv = yiv[pl.ds(k * 16, 16)]
        xl = xv - jnp.full((16,), 1, jnp.int32) * x0
        m = (fl == jnp.full((16,), 1, jnp.int32)) \
            & (xl >= jnp.full((16,), 0, jnp.int32)) \
            & (xl < jnp.full((16,), SEG_COLS, jnp.int32))
        cell = xl * jnp.full((16,), FM_H, jnp.int32) + yv
        cell = jnp.where(m, cell, jnp.full((16,), 0, jnp.int32))
        pvec = lanes + jnp.full((16,), 16, jnp.int32) * k
        for lm in lane_masks:
            plsc.store_scatter(mapv, [cell], pvec, mask=m & lm)
        return 0

    lax.fori_loop(0, NCHUNK, chunk_body, 0, unroll=2)

    moff = pl.multiple_of(b * HW + x0 * FM_H, 8)
    pltpu.sync_copy(mapv, pmap_hbm.at[pl.ds(moff, SEG_CELLS)])


def _compose_body(
    x_hbm, pmap_hbm, out_hbm, xv, mapv, outv, sm0, sm1, so0, so1
):
    wid = _wid()
    zero16 = jnp.full((16,), 0, jnp.int32)
    zf16 = jnp.full((16,), 0.0, jnp.float32)
    sm = (sm0, sm1)
    so = (so0, so1)

    def map_slice(b, s):
        soff = pl.multiple_of(b * HW + s * STRIPE_CELLS, 8)
        return pmap_hbm.at[pl.ds(soff, STRIPE_CELLS)]

    def out_slice(b, c0, c, s):
        row0 = pl.multiple_of((b * C + c0 + c) * FM_W + s * STRIPE_COLS, 8)
        return out_hbm.at[pl.ds(row0, STRIPE_COLS), :]

    for u in range(NUNIT_PER_TILE):
        g = wid * NUNIT_PER_TILE + u
        b = g // (C // CQ)
        c0 = (g % (C // CQ)) * CQ

        for par in range(2):
            pltpu.async_copy(map_slice(b, par), mapv.at[par], sm[par])

        xoff = pl.multiple_of((b * C + c0) * P, 8)
        pltpu.sync_copy(x_hbm.at[pl.ds(xoff, CQ * P)], xv)

        def sp_body(sp, _):
            for par in range(2):
                s = 2 * sp + par
                pltpu.make_async_copy(
                    map_slice(b, s), mapv.at[par], sm[par]
                ).wait()

                @pl.when(sp > 0)
                def _():
                    for c in range(CQ):
                        pltpu.make_async_copy(
                            outv.at[par, c],
                            out_slice(b, c0, c, s),
                            so[par],
                        ).wait()

                @plsc.parallel_loop(0, STRIPE_VECS, 1, unroll=8)
                def _(v):
                    w = lax.bitwise_and(v, STRIPE_COLS - 1)
                    h0 = lax.shift_left(
                        lax.shift_right_logical(v, STRIPE_COLS_LOG2), 4
                    )
                    m16 = mapv[par, pl.ds(w * FM_H + h0, 16)]
                    msk = m16 >= zero16
                    idx = jnp.maximum(m16, zero16)
                    for c in range(CQ):
                        gvals = plsc.load_gather(
                            xv, [jnp.full((16,), c * P, jnp.int32) + idx]
                        )
                        outv[par, c, w, pl.ds(h0, 16)] = jnp.where(
                            msk, gvals, zf16
                        )

                for c in range(CQ):
                    pltpu.async_copy(
                        outv.at[par, c],
                        out_slice(b, c0, c, s),
                        so[par],
                    )

                @pl.when(s + 2 < NSTRIPE)
                def _():
                    pltpu.async_copy(
                        map_slice(b, s + 2), mapv.at[par], sm[par]
                    )

            return 0

        lax.fori_loop(0, NSTRIPE // 2, sp_body, 0)

        for par in range(2):
            s_last = NSTRIPE - 2 + par
            for c in range(CQ):
                pltpu.make_async_copy(
                    outv.at[par, c],
                    out_slice(b, c0, c, s_last),
                    so[par],
                ).wait()


@functools.lru_cache(maxsize=1)
def _kernels():
    mesh = plsc.VectorSubcoreMesh(
        core_axis_name="c", subcore_axis_name="s", num_cores=2, num_subcores=16
    )
    params = pltpu.CompilerParams(needs_layout_passes=False)
    build_map = pl.kernel(
        _build_map_body,
        out_type=jax.ShapeDtypeStruct((B * HW,), jnp.int32),
        mesh=mesh,
        compiler_params=params,
        scratch_types=[
            pltpu.VMEM((P,), jnp.int32),
            pltpu.VMEM((P,), jnp.int32),
            pltpu.VMEM((P,), jnp.int32),
            pltpu.VMEM((SEG_CELLS,), jnp.int32),
            pltpu.SemaphoreType.DMA,
        ],
    )
    compose = pl.kernel(
        _compose_body,
        out_type=jax.ShapeDtypeStruct((B * C * FM_W, FM_H), jnp.float32),
        mesh=mesh,
        compiler_params=params,
        scratch_types=[
            pltpu.VMEM((CQ * P,), jnp.float32),
            pltpu.VMEM((2, STRIPE_CELLS), jnp.int32),
            pltpu.VMEM((2, CQ, STRIPE_COLS, FM_H), jnp.float32),
            pltpu.SemaphoreType.DMA,
            pltpu.SemaphoreType.DMA,
            pltpu.SemaphoreType.DMA,
            pltpu.SemaphoreType.DMA,
        ],
    )
    return build_map, compose


def kernel(x, inds):
    build_map, compose = _kernels()
    flag = inds[..., 0].astype(jnp.int32).reshape(-1)
    xi = inds[..., 1].astype(jnp.int32).reshape(-1)
    yi = inds[..., 2].astype(jnp.int32).reshape(-1)
    pmap = build_map(flag, xi, yi)
    out = compose(x.reshape(-1), pmap)
    return out.reshape(B, C, FM_W, FM_H).transpose(0, 1, 3, 2)
